# baseline (speedup 1.0000x reference)
import functools

import jax
import jax.numpy as jnp
from jax import lax
from jax.experimental import pallas as pl
from jax.experimental.pallas import tpu as pltpu
from jax.experimental.pallas import tpu_sc as plsc

N = 2048
G = 1.0
NC = 2
NS = 16
L = 16
NW = NC * NS
RPW = N // NW
NGRP = RPW // L

_F32 = jnp.float32
_MAGIC = jnp.int32(0x5F3759DF)


def _rsqrt16(r2):
    seed = plsc.bitcast(_MAGIC - (plsc.bitcast(r2, jnp.int32) >> 1), _F32)
    h = 0.5 * r2
    y = seed * (1.5 - h * seed * seed)
    y = y * (1.5 - h * y * y)
    return y


def _nbody_sc(xs_h, ys_h, zs_h, ms_h, pxs_h, pys_h, pzs_h,
              ox_h, oy_h, oz_h, opx_h, opy_h, opz_h,
              xv, yv, zv, mv, pxo, pyo, pzo,
              oxv, oyv, ozv, opxv, opyv, opzv):
    wid = lax.axis_index("s") * NC + lax.axis_index("c")
    base = wid * RPW

    pltpu.sync_copy(xs_h, xv)
    pltpu.sync_copy(ys_h, yv)
    pltpu.sync_copy(zs_h, zv)
    pltpu.sync_copy(ms_h, mv)
    pltpu.sync_copy(pxs_h.at[pl.ds(base, RPW)], pxo)
    pltpu.sync_copy(pys_h.at[pl.ds(base, RPW)], pyo)
    pltpu.sync_copy(pzs_h.at[pl.ds(base, RPW)], pzo)

    lane = lax.iota(jnp.int32, L)
    zeros = jnp.zeros((L,), _F32)

    for g in range(NGRP):
        gsl = pl.ds(g * L, L)
        xg = xv[pl.ds(base + g * L, L)]
        yg = yv[pl.ds(base + g * L, L)]
        zg = zv[pl.ds(base + g * L, L)]
        mg = mv[pl.ds(base + g * L, L)]

        def i_body(l, gacc, xg=xg, yg=yg, zg=zg, mg=mg):
            gx, gy, gz = gacc
            lmask = lane == l
            xi = jnp.full((L,), jnp.sum(jnp.where(lmask, xg, 0.0)))
            yi = jnp.full((L,), jnp.sum(jnp.where(lmask, yg, 0.0)))
            zi = jnp.full((L,), jnp.sum(jnp.where(lmask, zg, 0.0)))
            ci = jnp.full((L,), jnp.sum(jnp.where(lmask, G * mg, 0.0)))

            def j_body(c, acc, xi=xi, yi=yi, zi=zi, ci=ci):
                ax, ay, az = acc
                jsl = pl.ds(c * L, L)
                dx = xv[jsl] - xi
                dy = yv[jsl] - yi
                dz = zv[jsl] - zi
                mj = mv[jsl]
                r2 = dx * dx + dy * dy + dz * dz
                rinv = _rsqrt16(r2)
                w = ci * mj * (rinv * rinv * rinv)
                w = jnp.where(r2 > 0.0, w, 0.0)
                return (ax + w * dx, ay + w * dy, az + w * dz)

            ax, ay, az = lax.fori_loop(0, N // L, j_body,
                                       (zeros, zeros, zeros), unroll=4)
            gx = jnp.where(lmask, jnp.sum(ax), gx)
            gy = jnp.where(lmask, jnp.sum(ay), gy)
            gz = jnp.where(lmask, jnp.sum(az), gz)
            return (gx, gy, gz)

        gx, gy, gz = lax.fori_loop(0, L, i_body, (zeros, zeros, zeros))
        opxv[gsl] = gx
        opyv[gsl] = gy
        opzv[gsl] = gz
        minv = 1.0 / mg
        oxv[gsl] = pxo[gsl] * minv
        oyv[gsl] = pyo[gsl] * minv
        ozv[gsl] = pzo[gsl] * minv

    out_sl = pl.ds(base, RPW)
    pltpu.sync_copy(oxv, ox_h.at[out_sl])
    pltpu.sync_copy(oyv, oy_h.at[out_sl])
    pltpu.sync_copy(ozv, oz_h.at[out_sl])
    pltpu.sync_copy(opxv, opx_h.at[out_sl])
    pltpu.sync_copy(opyv, opy_h.at[out_sl])
    pltpu.sync_copy(opzv, opz_h.at[out_sl])


_sc_call = pl.kernel(
    _nbody_sc,
    out_type=[jax.ShapeDtypeStruct((N,), _F32)] * 6,
    mesh=plsc.VectorSubcoreMesh(core_axis_name="c", subcore_axis_name="s"),
    compiler_params=pltpu.CompilerParams(needs_layout_passes=False),
    scratch_types=(
        [pltpu.VMEM((N,), _F32)] * 4
        + [pltpu.VMEM((RPW,), _F32)] * 3
        + [pltpu.VMEM((RPW,), _F32)] * 6
    ),
)


def kernel(t, h, m, edge_index):
    d = h.shape[-1] // 2
    cols = [jnp.reshape(h[:, k], (N,)) for k in range(2 * d)]
    mm = jnp.reshape(m, (N,))
    outs = _sc_call(cols[0], cols[1], cols[2], mm,
                    cols[3], cols[4], cols[5])
    return jnp.stack(outs, axis=1)
